# Initial kernel scaffold; baseline (speedup 1.0000x reference)
import jax
import jax.numpy as jnp
from jax.experimental import pallas as pl


def kernel(document, table, W):
    raise NotImplementedError("write your pallas kernel here")


# speedup vs baseline: 2.3726x; 2.3726x over previous
import functools

import jax
import jax.numpy as jnp
from jax import lax
from jax.experimental import pallas as pl
from jax.experimental.pallas import tpu as pltpu
from jax.experimental.pallas import tpu_sc as plsc

BATCH = 16384
SEQ = 50
POOL = 20
DIM = 64

_INFO = plsc.get_sparse_core_info()
_NC = _INFO.num_cores
_NS = _INFO.num_subcores
_NW = _NC * _NS
_PER_W = BATCH // _NW
_C = 16
_G = _C // 4
_NCHUNK = _PER_W // _C


def _sc_pool(document, table):
    mesh = plsc.VectorSubcoreMesh(core_axis_name="c", subcore_axis_name="s")

    @functools.partial(
        pl.kernel,
        mesh=mesh,
        out_type=jax.ShapeDtypeStruct((BATCH, DIM), jnp.float32),
        compiler_params=pltpu.CompilerParams(use_tc_tiling_on_sc=False),
        scratch_types=[
            pltpu.VMEM((_C, SEQ), jnp.int32),
            pltpu.VMEM((_G, 80), jnp.int32),
            pltpu.VMEM((_G, 80, DIM), jnp.float32),
            pltpu.VMEM((_PER_W, DIM), jnp.float32),
            pltpu.SemaphoreType.DMA,
        ],
    )
    def k(doc_hbm, table_hbm, out_hbm, doc_v, idx_v, rows_v, pooled_v, sem):
        wid = lax.axis_index("s") * _NC + lax.axis_index("c")
        base = wid * _PER_W

        def chunk_body(c, carry):
            d0 = base + c * _C
            pltpu.sync_copy(doc_hbm.at[pl.ds(d0, _C), :], doc_v)
            for i in range(_C):
                g, s = i // 4, (i % 4) * 20
                lo = doc_v[i, pl.ds(0, 16)]
                hi = doc_v[i, pl.ds(4, 16)]
                idx_v[g, pl.ds(s, 16)] = lo
                idx_v[g, pl.ds(s + 4, 16)] = hi
            copies = [
                pltpu.async_copy(table_hbm.at[idx_v.at[g]], rows_v.at[g], sem)
                for g in range(_G)
            ]
            for cp in copies:
                cp.wait()
            for i in range(_C):
                g, r0 = i // 4, (i % 4) * 20
                for j in range(DIM // 16):
                    acc = rows_v[g, r0, pl.ds(j * 16, 16)]
                    for t in range(1, POOL):
                        acc = acc + rows_v[g, r0 + t, pl.ds(j * 16, 16)]
                    pooled_v[c * _C + i, pl.ds(j * 16, 16)] = acc
            return carry

        lax.fori_loop(0, _NCHUNK, chunk_body, 0)
        pltpu.sync_copy(pooled_v, out_hbm.at[pl.ds(base, _PER_W), :])

    return k(document, table)


def _tc_linear(pooled, W):
    blk = 1024

    def mm(x_ref, w_ref, o_ref):
        o_ref[...] = lax.dot_general(
            x_ref[...], w_ref[...],
            dimension_numbers=(((1,), (1,)), ((), ())),
            preferred_element_type=jnp.float32,
        )

    return pl.pallas_call(
        mm,
        grid=(BATCH // blk,),
        in_specs=[
            pl.BlockSpec((blk, DIM), lambda i: (i, 0)),
            pl.BlockSpec((DIM, DIM), lambda i: (0, 0)),
        ],
        out_specs=pl.BlockSpec((blk, DIM), lambda i: (i, 0)),
        out_shape=jax.ShapeDtypeStruct((BATCH, DIM), jnp.float32),
    )(pooled, W)


def kernel(document, table, W):
    document = document.astype(jnp.int32)
    pooled = _sc_pool(document, table)
    return _tc_linear(pooled, W)


# speedup vs baseline: 2.5026x; 1.0548x over previous
import functools

import jax
import jax.numpy as jnp
from jax import lax
from jax.experimental import pallas as pl
from jax.experimental.pallas import tpu as pltpu
from jax.experimental.pallas import tpu_sc as plsc

BATCH = 16384
SEQ = 50
POOL = 20
DIM = 64

_INFO = plsc.get_sparse_core_info()
_NC = _INFO.num_cores
_NS = _INFO.num_subcores
_NW = _NC * _NS
_PER_W = BATCH // _NW
_C = 16
_G = _C // 4
_NCHUNK = _PER_W // _C


def _sc_pool(document, table):
    mesh = plsc.VectorSubcoreMesh(core_axis_name="c", subcore_axis_name="s")

    @functools.partial(
        pl.kernel,
        mesh=mesh,
        out_type=jax.ShapeDtypeStruct((BATCH, DIM), jnp.float32),
        compiler_params=pltpu.CompilerParams(use_tc_tiling_on_sc=False),
        scratch_types=[
            pltpu.VMEM((_PER_W, SEQ), jnp.int32),
            pltpu.VMEM((_NCHUNK * _G, 80), jnp.int32),
            pltpu.VMEM((2, _G, 80, DIM), jnp.float32),
            pltpu.VMEM((_PER_W, DIM), jnp.float32),
            pltpu.SemaphoreType.DMA,
            pltpu.SemaphoreType.DMA,
        ],
    )
    def k(doc_hbm, table_hbm, out_hbm, doc_v, idx_v, rows_v, pooled_v,
          sem0, sem1):
        wid = lax.axis_index("s") * _NC + lax.axis_index("c")
        base = wid * _PER_W
        sems = (sem0, sem1)

        pltpu.sync_copy(doc_hbm.at[pl.ds(base, _PER_W), :], doc_v)

        def compact_body(r, carry):
            for s in range(4):
                i = r * 4 + s
                lo = doc_v[i, pl.ds(0, 16)]
                hi = doc_v[i, pl.ds(4, 16)]
                idx_v[r, pl.ds(s * 20, 16)] = lo
                idx_v[r, pl.ds(s * 20 + 4, 16)] = hi
            return carry

        lax.fori_loop(0, _PER_W // 4, compact_body, 0)

        def fire(c, p):
            for g in range(_G):
                pltpu.async_copy(
                    table_hbm.at[idx_v.at[c * _G + g]], rows_v.at[p, g],
                    sems[p])

        def drain(c, p):
            for g in range(_G):
                pltpu.make_async_copy(
                    table_hbm.at[idx_v.at[c * _G + g]], rows_v.at[p, g],
                    sems[p]).wait()

        fire(0, 0)
        fire(1, 1)

        def pair_body(h, carry):
            for p in range(2):
                c = h * 2 + p
                drain(c, p)
                for i in range(_C):
                    g, r0 = i // 4, (i % 4) * 20
                    for j in range(DIM // 16):
                        acc = rows_v[p, g, r0, pl.ds(j * 16, 16)]
                        for t in range(1, POOL):
                            acc = acc + rows_v[p, g, r0 + t, pl.ds(j * 16, 16)]
                        pooled_v[c * _C + i, pl.ds(j * 16, 16)] = acc

                @pl.when(c + 2 < _NCHUNK)
                def _():
                    fire(c + 2, p)

            return carry

        lax.fori_loop(0, _NCHUNK // 2, pair_body, 0)
        pltpu.sync_copy(pooled_v, out_hbm.at[pl.ds(base, _PER_W), :])

    return k(document, table)


def _tc_linear(pooled, W):
    blk = 1024

    def mm(x_ref, w_ref, o_ref):
        o_ref[...] = lax.dot_general(
            x_ref[...], w_ref[...],
            dimension_numbers=(((1,), (1,)), ((), ())),
            preferred_element_type=jnp.float32,
        )

    return pl.pallas_call(
        mm,
        grid=(BATCH // blk,),
        in_specs=[
            pl.BlockSpec((blk, DIM), lambda i: (i, 0)),
            pl.BlockSpec((DIM, DIM), lambda i: (0, 0)),
        ],
        out_specs=pl.BlockSpec((blk, DIM), lambda i: (i, 0)),
        out_shape=jax.ShapeDtypeStruct((BATCH, DIM), jnp.float32),
    )(pooled, W)


def kernel(document, table, W):
    document = document.astype(jnp.int32)
    pooled = _sc_pool(document, table)
    return _tc_linear(pooled, W)
